# baseline (speedup 1.0000x reference)
import functools

import jax
import jax.numpy as jnp
from jax import lax
from jax.experimental import pallas as pl
from jax.experimental.pallas import tpu as pltpu
from jax.experimental.pallas import tpu_sc as plsc

N_NODES = 10000
N_EDGES = 160000
D = 256
DH = 128
DE = 16
DEA = 32
NC = 2
NS = 16
L = 16

GK = 80
NCHUNK = 2048
E_PAD = NCHUNK * GK
CPT = NCHUNK // NS
GBLK = 8
NBLK = NCHUNK // GBLK
BPT = NBLK // NS
NRING = 4
ECPT = NCHUNK // (NC * NS)
PAD_ROW = N_NODES
N_PAD = 10112
ROWS_PER_TILE = N_PAD // NS

_MESH = plsc.VectorSubcoreMesh(core_axis_name="c", subcore_axis_name="s")


def _zero_fill(buf, rows, width):
    zero16 = jnp.zeros((L,), jnp.float32)

    def fill(i, carry):
        for j in range(width // L):
            buf[i, pl.ds(j * L, L)] = zero16
        return carry

    lax.fori_loop(0, rows, fill, 0, unroll=4)


def _init_acc(zeros_buf, nrows, acc, row0):
    for t in range(ROWS_PER_TILE // nrows):
        pltpu.sync_copy(zeros_buf, acc.at[pl.ds(row0 + t * nrows, nrows)])
    rem = ROWS_PER_TILE % nrows
    if rem:
        pltpu.sync_copy(
            zeros_buf.at[pl.ds(0, rem)],
            acc.at[pl.ds(row0 + (ROWS_PER_TILE // nrows) * nrows, rem)])


def _sc_g_body(xs_hbm, idx_hbm, g_out,
               ia, ib, r0, r1, r2, r3, acc_g, s0, s1, s2, s3):
    c = lax.axis_index("c")
    s = lax.axis_index("s")
    rows = (r0, r1, r2, r3)
    sems = (s0, s1, s2, s3)

    _zero_fill(r0, GK, DH)
    row0 = s * ROWS_PER_TILE
    _init_acc(r0, GK, acc_g, row0)
    plsc.subcore_barrier()

    b0 = s * BPT
    pltpu.sync_copy(idx_hbm.at[c, b0], ia)
    for i in range(NRING - 1):
        pltpu.async_copy(xs_hbm.at[ia.at[i]], rows[i], sems[i])

    def body(j, carry):
        blk = b0 + 2 * j
        pltpu.sync_copy(idx_hbm.at[c, blk + 1], ib)
        for t in range(2 * GBLK):
            if t == GBLK:
                @pl.when(j < BPT // 2 - 1)
                def _():
                    pltpu.sync_copy(idx_hbm.at[c, blk + 2], ia)
            k = t % NRING
            pltpu.make_async_copy(xs_hbm.at[pl.ds(0, GK)], rows[k],
                                  sems[k]).wait()
            ridx = ia.at[GBLK + t] if t < GBLK else ib.at[t]
            pltpu.sync_copy(rows[k], acc_g.at[ridx], add=True)
            nt = t + NRING - 1
            if nt < GBLK:
                gidx = ia.at[nt]
            elif nt < 2 * GBLK:
                gidx = ib.at[nt - GBLK]
            else:
                gidx = ia.at[nt - 2 * GBLK]
            kn = nt % NRING
            pltpu.async_copy(xs_hbm.at[gidx], rows[kn], sems[kn])
        return carry

    lax.fori_loop(0, BPT // 2, body, 0)
    for i in range(NRING - 1):
        k = i % NRING
        pltpu.make_async_copy(xs_hbm.at[pl.ds(0, GK)], rows[k],
                              sems[k]).wait()
    plsc.subcore_barrier()

    pltpu.sync_copy(acc_g.at[pl.ds(row0, ROWS_PER_TILE)],
                    g_out.at[c, pl.ds(row0, ROWS_PER_TILE)])


_sc_g = functools.partial(
    pl.kernel,
    out_type=jax.ShapeDtypeStruct((NC, N_PAD, DH), jnp.float32),
    mesh=_MESH,
    scratch_types=[
        pltpu.VMEM((2 * GBLK, GK), jnp.int32),
        pltpu.VMEM((2 * GBLK, GK), jnp.int32),
        pltpu.VMEM((GK, DH), jnp.float32),
        pltpu.VMEM((GK, DH), jnp.float32),
        pltpu.VMEM((GK, DH), jnp.float32),
        pltpu.VMEM((GK, DH), jnp.float32),
        pltpu.VMEM_SHARED((N_PAD, DH), jnp.float32),
        pltpu.SemaphoreType.DMA,
        pltpu.SemaphoreType.DMA,
        pltpu.SemaphoreType.DMA,
        pltpu.SemaphoreType.DMA,
    ],
)(_sc_g_body)


def _sc_e_body(ea_hbm, rcv_hbm, e_out,
               rcv_v, ea_st, erows, acc_e):
    c = lax.axis_index("c")
    s = lax.axis_index("s")

    _zero_fill(erows, GK, DH)
    row0 = s * ROWS_PER_TILE
    _init_acc(erows, GK, acc_e, row0)

    one16 = jnp.where(lax.iota(jnp.int32, L) == 0,
                      jnp.float32(1.0), jnp.float32(0.0))

    def ones_fill(r, carry):
        erows[r, pl.ds(DE, L)] = one16
        return carry

    lax.fori_loop(0, GK, ones_fill, 0, unroll=4)

    w = c * NS + s
    e0 = w * ECPT
    pltpu.sync_copy(rcv_hbm.at[pl.ds(e0, ECPT)], rcv_v)
    plsc.subcore_barrier()

    half = ECPT // 4
    rows_per_chunk = GK // 8
    stage_rows = half * rows_per_chunk
    for h in range(4):
        pltpu.sync_copy(
            ea_hbm.at[pl.ds((e0 + h * half) * rows_per_chunk, stage_rows)],
            ea_st)

        def echunk(kk, carry):
            r0 = kk * rows_per_chunk

            def fill(rr, carry2):
                for q in range(8):
                    erows[rr * 8 + q, pl.ds(0, L)] = (
                        ea_st[r0 + rr, pl.ds(q * L, L)])
                return carry2

            lax.fori_loop(0, rows_per_chunk, fill, 0, unroll=4)
            pltpu.sync_copy(erows, acc_e.at[rcv_v.at[h * half + kk]],
                            add=True)
            return carry

        lax.fori_loop(0, half, echunk, 0)
    plsc.subcore_barrier()

    pltpu.sync_copy(acc_e.at[pl.ds(row0, ROWS_PER_TILE)],
                    e_out.at[c, pl.ds(row0, ROWS_PER_TILE)])


_sc_e = functools.partial(
    pl.kernel,
    out_type=jax.ShapeDtypeStruct((NC, N_PAD, DH), jnp.float32),
    mesh=_MESH,
    scratch_types=[
        pltpu.VMEM((ECPT, GK), jnp.int32),
        pltpu.VMEM((ECPT // 4 * (GK // 8), 8 * DE), jnp.float32),
        pltpu.VMEM((GK, DH), jnp.float32),
        pltpu.VMEM_SHARED((N_PAD, DH), jnp.float32),
    ],
)(_sc_e_body)


def _tc_body(x_ref, g_ref, ea_ref, wm_ref, we_ref, bm_ref, be_ref, wu_ref,
             bu_ref, o_ref):
    f32 = jnp.float32
    ea = ea_ref[0] + ea_ref[1]
    t = jnp.dot(g_ref[0], wm_ref[:DH, :], preferred_element_type=f32)
    t = t + jnp.dot(g_ref[1], wm_ref[DH:, :], preferred_element_type=f32)
    t = t + jnp.dot(ea[:, :DE], we_ref[...], preferred_element_type=f32)
    t = t + ea[:, DE:DE + 1] * (bm_ref[...] + be_ref[...])
    u = jnp.dot(x_ref[...], wu_ref[:D, :], preferred_element_type=f32)
    u = u + jnp.dot(t, wu_ref[D:, :], preferred_element_type=f32)
    u = u + bu_ref[...]
    o_ref[...] = jnp.maximum(u, 0.0)


ROW_BLK = 1000

_tc_update = pl.pallas_call(
    _tc_body,
    grid=(N_NODES // ROW_BLK,),
    in_specs=[
        pl.BlockSpec((ROW_BLK, D), lambda i: (i, 0)),
        pl.BlockSpec((NC, ROW_BLK, DH), lambda i: (0, i, 0)),
        pl.BlockSpec((NC, ROW_BLK, DH), lambda i: (0, i, 0)),
        pl.BlockSpec((D, D), lambda i: (0, 0)),
        pl.BlockSpec((DE, D), lambda i: (0, 0)),
        pl.BlockSpec((1, D), lambda i: (0, 0)),
        pl.BlockSpec((1, D), lambda i: (0, 0)),
        pl.BlockSpec((2 * D, D), lambda i: (0, 0)),
        pl.BlockSpec((1, D), lambda i: (0, 0)),
    ],
    out_specs=pl.BlockSpec((ROW_BLK, D), lambda i: (i, 0)),
    out_shape=jax.ShapeDtypeStruct((N_NODES, D), jnp.float32),
)


def kernel(x, edge_attr, W_msg, b_msg, W_edge, b_edge, W_upd, b_upd, senders,
           receivers):
    xs = jnp.concatenate([x[:, :DH], x[:, DH:]], axis=0)
    npad = E_PAD - N_EDGES
    snd_p = jnp.concatenate([senders, jnp.zeros((npad,), jnp.int32)])
    rcv_p = jnp.concatenate(
        [receivers, jnp.full((npad,), PAD_ROW, jnp.int32)])
    snd_r = snd_p.reshape(NBLK, GBLK, GK)
    rcv_r = rcv_p.reshape(NBLK, GBLK, GK)
    blk0 = jnp.concatenate([snd_r, rcv_r], axis=1)
    blk1 = jnp.concatenate([snd_r + N_NODES, rcv_r], axis=1)
    idx4 = jnp.stack([blk0, blk1])
    rcv2 = rcv_p.reshape(NCHUNK, GK)
    ea2 = jnp.concatenate(
        [edge_attr, jnp.zeros((npad, DE), jnp.float32)],
        axis=0).reshape(E_PAD // 8, 8 * DE)
    g = _sc_g(xs, idx4)
    ep = _sc_e(ea2, rcv2)
    return _tc_update(x, g, ep, W_msg, W_edge,
                      b_msg.reshape(1, D), b_edge.reshape(1, D),
                      W_upd, b_upd.reshape(1, D))


# speedup vs baseline: 1.0601x; 1.0601x over previous
import functools

import jax
import jax.numpy as jnp
from jax import lax
from jax.experimental import pallas as pl
from jax.experimental.pallas import tpu as pltpu
from jax.experimental.pallas import tpu_sc as plsc

N_NODES = 10000
N_EDGES = 160000
D = 256
DH = 128
DE = 16
DEA = 32
NC = 2
NS = 16
L = 16

GK = 128
NCHUNK = 1280
E_PAD = NCHUNK * GK
CPT = NCHUNK // NS
GBLK = 8
NBLK = NCHUNK // GBLK
BPT = NBLK // NS
ECPT = NCHUNK // (NC * NS)
PAD_ROW = N_NODES
N_PAD = 10112
ROWS_PER_TILE = N_PAD // NS

_MESH = plsc.VectorSubcoreMesh(core_axis_name="c", subcore_axis_name="s")


def _zero_fill(buf, rows, width):
    zero16 = jnp.zeros((L,), jnp.float32)

    def fill(i, carry):
        for j in range(width // L):
            buf[i, pl.ds(j * L, L)] = zero16
        return carry

    lax.fori_loop(0, rows, fill, 0, unroll=4)


def _init_acc(zeros_buf, acc, row0):
    for t in range(ROWS_PER_TILE // GK):
        pltpu.sync_copy(zeros_buf, acc.at[pl.ds(row0 + t * GK, GK)])
    rem = ROWS_PER_TILE % GK
    pltpu.sync_copy(zeros_buf.at[pl.ds(0, rem)],
                    acc.at[pl.ds(row0 + (ROWS_PER_TILE // GK) * GK, rem)])


def _sc_g_body(xs_hbm, idx_hbm, g_out,
               ia, ib, rows_a, rows_b, acc_g, sem_a, sem_b):
    c = lax.axis_index("c")
    s = lax.axis_index("s")

    _zero_fill(rows_a, GK, DH)
    row0 = s * ROWS_PER_TILE
    _init_acc(rows_a, acc_g, row0)
    plsc.subcore_barrier()

    b0 = s * BPT
    pltpu.sync_copy(idx_hbm.at[c, b0], ia)
    pltpu.async_copy(xs_hbm.at[ia.at[0]], rows_a, sem_a)
    pltpu.async_copy(xs_hbm.at[ia.at[1]], rows_b, sem_b)

    def pair(j, carry):
        blk = b0 + 2 * j
        pltpu.sync_copy(idx_hbm.at[c, blk + 1], ib)
        for t in range(0, GBLK, 2):
            pltpu.make_async_copy(xs_hbm.at[pl.ds(0, GK)], rows_a,
                                  sem_a).wait()
            pltpu.sync_copy(rows_a, acc_g.at[ia.at[GBLK + t]], add=True)
            src = ia.at[t + 2] if t < GBLK - 2 else ib.at[0]
            pltpu.async_copy(xs_hbm.at[src], rows_a, sem_a)
            pltpu.make_async_copy(xs_hbm.at[pl.ds(0, GK)], rows_b,
                                  sem_b).wait()
            pltpu.sync_copy(rows_b, acc_g.at[ia.at[GBLK + t + 1]], add=True)
            src = ia.at[t + 3] if t < GBLK - 2 else ib.at[1]
            pltpu.async_copy(xs_hbm.at[src], rows_b, sem_b)

        @pl.when(j < BPT // 2 - 1)
        def _():
            pltpu.sync_copy(idx_hbm.at[c, blk + 2], ia)

        for t in range(0, GBLK, 2):
            pltpu.make_async_copy(xs_hbm.at[pl.ds(0, GK)], rows_a,
                                  sem_a).wait()
            pltpu.sync_copy(rows_a, acc_g.at[ib.at[GBLK + t]], add=True)
            src = ib.at[t + 2] if t < GBLK - 2 else ia.at[0]
            pltpu.async_copy(xs_hbm.at[src], rows_a, sem_a)
            pltpu.make_async_copy(xs_hbm.at[pl.ds(0, GK)], rows_b,
                                  sem_b).wait()
            pltpu.sync_copy(rows_b, acc_g.at[ib.at[GBLK + t + 1]], add=True)
            src = ib.at[t + 3] if t < GBLK - 2 else ia.at[1]
            pltpu.async_copy(xs_hbm.at[src], rows_b, sem_b)
        return carry

    lax.fori_loop(0, BPT // 2, pair, 0)
    pltpu.make_async_copy(xs_hbm.at[pl.ds(0, GK)], rows_a, sem_a).wait()
    pltpu.make_async_copy(xs_hbm.at[pl.ds(0, GK)], rows_b, sem_b).wait()
    plsc.subcore_barrier()

    pltpu.sync_copy(acc_g.at[pl.ds(row0, ROWS_PER_TILE)],
                    g_out.at[c, pl.ds(row0, ROWS_PER_TILE)])


_sc_g = functools.partial(
    pl.kernel,
    out_type=jax.ShapeDtypeStruct((NC, N_PAD, DH), jnp.float32),
    mesh=_MESH,
    scratch_types=[
        pltpu.VMEM((2 * GBLK, GK), jnp.int32),
        pltpu.VMEM((2 * GBLK, GK), jnp.int32),
        pltpu.VMEM((GK, DH), jnp.float32),
        pltpu.VMEM((GK, DH), jnp.float32),
        pltpu.VMEM_SHARED((N_PAD, DH), jnp.float32),
        pltpu.SemaphoreType.DMA,
        pltpu.SemaphoreType.DMA,
    ],
)(_sc_g_body)


def _sc_e_body(ea_hbm, rcv_hbm, e_out,
               rcv_v, ea_st, er_a, er_b, acc_e, sem_a, sem_b):
    c = lax.axis_index("c")
    s = lax.axis_index("s")

    _zero_fill(er_a, GK, DH)
    _zero_fill(er_b, GK, DH)
    row0 = s * ROWS_PER_TILE
    _init_acc(er_a, acc_e, row0)

    one16 = jnp.where(lax.iota(jnp.int32, L) == 0,
                      jnp.float32(1.0), jnp.float32(0.0))

    def ones_fill(r, carry):
        er_a[r, pl.ds(DE, L)] = one16
        er_b[r, pl.ds(DE, L)] = one16
        return carry

    lax.fori_loop(0, GK, ones_fill, 0, unroll=4)

    w = c * NS + s
    e0 = w * ECPT
    pltpu.sync_copy(rcv_hbm.at[pl.ds(e0, ECPT)], rcv_v)
    plsc.subcore_barrier()

    rpc = GK // 8
    stage = 5
    ebufs = (er_a, er_b)
    esems = (sem_a, sem_b)

    def expand_into(buf, r0):
        def fill(rr, carry):
            for q in range(8):
                buf[rr * 8 + q, pl.ds(0, L)] = ea_st[r0 + rr, pl.ds(q * L, L)]
            return carry

        lax.fori_loop(0, rpc, fill, 0, unroll=4)

    def wait_scatter(k):
        pltpu.make_async_copy(ea_hbm.at[pl.ds(0, GK)], ebufs[k],
                              esems[k]).wait()

    def do_pair(p, skip_first_waits):
        for hh in range(2):
            h = 2 * p + hh
            pltpu.sync_copy(
                ea_hbm.at[pl.ds((e0 + h * stage) * rpc, stage * rpc)],
                ea_st)
            for kk in range(stage):
                j = stage * hh + kk
                k = j % 2
                if not (skip_first_waits and j < 2):
                    wait_scatter(k)
                expand_into(ebufs[k], kk * rpc)
                pltpu.async_copy(ebufs[k], acc_e.at[rcv_v.at[h * stage + kk]],
                                 esems[k], add=True)

    do_pair(0, True)

    def pair(p, carry):
        do_pair(p, False)
        return carry

    lax.fori_loop(1, ECPT // (2 * stage), pair, 0)
    wait_scatter(0)
    wait_scatter(1)
    plsc.subcore_barrier()

    pltpu.sync_copy(acc_e.at[pl.ds(row0, ROWS_PER_TILE)],
                    e_out.at[c, pl.ds(row0, ROWS_PER_TILE)])


_sc_e = functools.partial(
    pl.kernel,
    out_type=jax.ShapeDtypeStruct((NC, N_PAD, DH), jnp.float32),
    mesh=_MESH,
    scratch_types=[
        pltpu.VMEM((ECPT, GK), jnp.int32),
        pltpu.VMEM((5 * (GK // 8), GK), jnp.float32),
        pltpu.VMEM((GK, DH), jnp.float32),
        pltpu.VMEM((GK, DH), jnp.float32),
        pltpu.VMEM_SHARED((N_PAD, DH), jnp.float32),
        pltpu.SemaphoreType.DMA,
        pltpu.SemaphoreType.DMA,
    ],
)(_sc_e_body)


def _tc_body(x_ref, g_ref, ea_ref, wm_ref, we_ref, bm_ref, be_ref, wu_ref,
             bu_ref, o_ref):
    f32 = jnp.float32
    ea = ea_ref[0] + ea_ref[1]
    t = jnp.dot(g_ref[0], wm_ref[:DH, :], preferred_element_type=f32)
    t = t + jnp.dot(g_ref[1], wm_ref[DH:, :], preferred_element_type=f32)
    t = t + jnp.dot(ea[:, :DE], we_ref[...], preferred_element_type=f32)
    t = t + ea[:, DE:DE + 1] * (bm_ref[...] + be_ref[...])
    u = jnp.dot(x_ref[...], wu_ref[:D, :], preferred_element_type=f32)
    u = u + jnp.dot(t, wu_ref[D:, :], preferred_element_type=f32)
    u = u + bu_ref[...]
    o_ref[...] = jnp.maximum(u, 0.0)


ROW_BLK = 1000

_tc_update = pl.pallas_call(
    _tc_body,
    grid=(N_NODES // ROW_BLK,),
    in_specs=[
        pl.BlockSpec((ROW_BLK, D), lambda i: (i, 0)),
        pl.BlockSpec((NC, ROW_BLK, DH), lambda i: (0, i, 0)),
        pl.BlockSpec((NC, ROW_BLK, DH), lambda i: (0, i, 0)),
        pl.BlockSpec((D, D), lambda i: (0, 0)),
        pl.BlockSpec((DE, D), lambda i: (0, 0)),
        pl.BlockSpec((1, D), lambda i: (0, 0)),
        pl.BlockSpec((1, D), lambda i: (0, 0)),
        pl.BlockSpec((2 * D, D), lambda i: (0, 0)),
        pl.BlockSpec((1, D), lambda i: (0, 0)),
    ],
    out_specs=pl.BlockSpec((ROW_BLK, D), lambda i: (i, 0)),
    out_shape=jax.ShapeDtypeStruct((N_NODES, D), jnp.float32),
)


def kernel(x, edge_attr, W_msg, b_msg, W_edge, b_edge, W_upd, b_upd, senders,
           receivers):
    xs = jnp.concatenate([x[:, :DH], x[:, DH:]], axis=0)
    npad = E_PAD - N_EDGES
    snd_p = jnp.concatenate([senders, jnp.zeros((npad,), jnp.int32)])
    rcv_p = jnp.concatenate(
        [receivers, jnp.full((npad,), PAD_ROW, jnp.int32)])
    snd_r = snd_p.reshape(NBLK, GBLK, GK)
    rcv_r = rcv_p.reshape(NBLK, GBLK, GK)
    blk0 = jnp.concatenate([snd_r, rcv_r], axis=1)
    blk1 = jnp.concatenate([snd_r + N_NODES, rcv_r], axis=1)
    idx4 = jnp.stack([blk0, blk1])
    rcv2 = rcv_p.reshape(NCHUNK, GK)
    ea2 = jnp.concatenate(
        [edge_attr, jnp.zeros((npad, DE), jnp.float32)],
        axis=0).reshape(E_PAD // 8, 8 * DE)
    g = _sc_g(xs, idx4)
    ep = _sc_e(ea2, rcv2)
    return _tc_update(x, g, ep, W_msg, W_edge,
                      b_msg.reshape(1, D), b_edge.reshape(1, D),
                      W_upd, b_upd.reshape(1, D))
